# baseline (speedup 1.0000x reference)
import functools

import jax
import jax.numpy as jnp
from jax import lax
from jax.experimental import pallas as pl
from jax.experimental.pallas import tpu as pltpu
from jax.experimental.pallas import tpu_sc as plsc

_CHUNK = 2048


def _vq_argmin_kernel(zs_ref, z2b_ref, et_ref, es_ref, idx_ref, dsum_ref):
    c = lax.dot_general(
        z2b_ref[...], et_ref[...],
        dimension_numbers=(((1,), (0,)), ((), ())),
        preferred_element_type=jnp.float32,
    )
    d2 = (zs_ref[...] - c) + es_ref[...]
    dist = jnp.sqrt(jnp.maximum(d2, 0.0))
    br, v = dist.shape
    w = _CHUNK if v % _CHUNK == 0 else v
    nch = v // w
    run_v = jnp.full((br, 1), jnp.inf, jnp.float32)
    run_i = jnp.zeros((br, 1), jnp.int32)
    gmin = jnp.full((br, 1), jnp.inf, jnp.float32)
    for k in range(nch):
        sub = lax.slice_in_dim(dist, k * w, (k + 1) * w, axis=1)
        m = jnp.min(sub, axis=1, keepdims=True)
        iota = lax.broadcasted_iota(jnp.int32, sub.shape, 1) + k * w
        i = jnp.min(jnp.where(sub == m, iota, v), axis=1, keepdims=True)
        keep = jnp.logical_or(run_v < m,
                              jnp.logical_and(run_v == m, run_i < i))
        run_i = jnp.where(keep, run_i, i)
        mv = m.astype(jnp.bfloat16).astype(jnp.float32)
        run_v = jnp.where(keep, run_v, mv)
        gmin = jnp.minimum(gmin, m)
    idx_ref[0, 0, :] = run_i[:, 0]
    gm = gmin[:, 0]
    dsum_ref[0, 0, 0] = jnp.sum(gm * gm)


def _vq_argmin(z2b, zs, et, es, block_rows):
    n, d = z2b.shape
    v = et.shape[1]
    nb = n // block_rows
    idx3, dsum = pl.pallas_call(
        _vq_argmin_kernel,
        grid=(nb,),
        in_specs=[
            pl.BlockSpec((block_rows, 1), lambda i: (i, 0)),
            pl.BlockSpec((block_rows, d), lambda i: (i, 0)),
            pl.BlockSpec((d, v), lambda i: (0, 0)),
            pl.BlockSpec((1, v), lambda i: (0, 0)),
        ],
        out_specs=[
            pl.BlockSpec((1, 1, block_rows), lambda i: (i, 0, 0)),
            pl.BlockSpec((1, 1, 1), lambda i: (i, 0, 0),
                         memory_space=pltpu.SMEM),
        ],
        out_shape=[
            jax.ShapeDtypeStruct((nb, 1, block_rows), jnp.int32),
            jax.ShapeDtypeStruct((nb, 1, 1), jnp.float32),
        ],
    )(zs, z2b, et, es)
    return idx3.reshape(n), dsum.reshape(nb)


def _sc_gather(table, idx):
    b = idx.shape[0]
    v, d = table.shape
    info = plsc.get_sparse_core_info()
    nw = info.num_cores * info.num_subcores
    b_per_w = b // nw
    ch = 128 if b_per_w % 128 == 0 else b_per_w
    nch = b_per_w // ch
    mesh = plsc.VectorSubcoreMesh(core_axis_name="c", subcore_axis_name="s")

    @functools.partial(
        pl.kernel,
        mesh=mesh,
        out_type=jax.ShapeDtypeStruct((b, d), jnp.float32),
        scratch_types=[
            pltpu.VMEM((nch, ch), jnp.int32),
            pltpu.VMEM((ch, d), jnp.float32),
            pltpu.SemaphoreType.DMA,
        ],
    )
    def k(table_hbm, idx_hbm, out_hbm, idx_v, rows_v, sem):
        wid = lax.axis_index("s") * info.num_cores + lax.axis_index("c")
        base = wid * b_per_w
        for c_i in range(nch):
            off = base + c_i * ch
            pltpu.sync_copy(idx_hbm.at[pl.ds(off, ch)], idx_v.at[c_i])
            pltpu.async_copy(table_hbm.at[idx_v.at[c_i]], rows_v, sem).wait()
            pltpu.sync_copy(rows_v, out_hbm.at[pl.ds(off, ch)])

    return k(table, idx)


def kernel(z, embedding):
    b, d = z.shape[0], z.shape[1]
    spatial = z.shape[2:]
    z_flat = z.reshape(b, d, -1).transpose(0, 2, 1).reshape(-1, d)
    zs = jnp.sum(z_flat ** 2, axis=1, keepdims=True)
    es = jnp.sum(embedding ** 2, axis=1)[None, :]
    z2b = (2.0 * z_flat).astype(jnp.bfloat16)

    indices_flat, dsum = _vq_argmin(z2b, zs, embedding.T, es, block_rows=256)
    dpad = (-d) % 128
    table = jnp.pad(embedding, ((0, 0), (0, dpad))) if dpad else embedding
    quantized_flat = _sc_gather(table, indices_flat)[:, :d]

    quantized = (quantized_flat.reshape(b, -1, d)
                 .transpose(0, 2, 1).reshape(b, d, *spatial))
    commit_loss = jnp.sum(dsum) / z.size
    quantized_st = z + lax.stop_gradient(quantized - z)
    indices = indices_flat.reshape(b, -1)
    return quantized_st, indices, commit_loss
